# baseline (speedup 1.0000x reference)
import jax
import jax.numpy as jnp
from jax import lax
from jax.experimental import pallas as pl
from jax.experimental.pallas import tpu as pltpu
from jax.experimental.pallas import tpu_sc as plsc

N = 50000
NP = 50048
HALF = NP // 2
ACC5 = 25088
D = 64
E = 1_000_000
EP = 1 << 20
PT = 32
PT_E = EP // PT
PT_ROWS = PT_E // 128
CAP = 17408
CROWS = CAP // 128
GROWS = PT * CROWS
TR5 = 2 * CROWS
SB5 = 16
NSB5 = TR5 // SB5
OPT = NP // 16
TROWS = (EP // 128) // 16
SBD = 16
NSBD = TROWS // SBD
LAYERS = 3

_mesh = plsc.VectorSubcoreMesh(core_axis_name="c", subcore_axis_name="s")


def _part_body(srcr, dstr, ga, sa, gb, sb_, bufk, bufv, gb0, sb0, gb1, sb1):
    c = lax.axis_index("c")
    s = lax.axis_index("s")
    w = c * 16 + s

    def run_dir(keyr, valr, gout, sout):
        def dfl(i, _):
            z = jnp.zeros((16,), jnp.int32)
            t = jnp.full((16,), HALF, jnp.int32)
            gb0[pl.ds(i * 16, 16)] = z
            gb1[pl.ds(i * 16, 16)] = z
            sb0[pl.ds(i * 16, 16)] = t
            sb1[pl.ds(i * 16, 16)] = t
            return 0

        lax.fori_loop(0, CAP // 16, dfl, 0)

        def chunk(k, ptrs):
            pltpu.sync_copy(keyr.at[pl.ds(w * PT_ROWS + k * 4, 4)], bufk)
            pltpu.sync_copy(valr.at[pl.ds(w * PT_ROWS + k * 4, 4)], bufv)

            def grp(q, ptrs2):
                w0, w1 = ptrs2
                for r in range(4):
                    kv = bufk[r, pl.ds(16 * q, 16)]
                    vv = bufv[r, pl.ds(16 * q, 16)]
                    m0 = kv < HALF
                    m1 = jnp.logical_and(kv >= HALF, kv < N)
                    plsc.store_compressed(gb0.at[pl.ds(w0, 16)], vv, mask=m0)
                    plsc.store_compressed(sb0.at[pl.ds(w0, 16)], kv, mask=m0)
                    plsc.store_compressed(gb1.at[pl.ds(w1, 16)], vv, mask=m1)
                    plsc.store_compressed(
                        sb1.at[pl.ds(w1, 16)], kv - HALF, mask=m1)
                    pc0 = plsc.all_reduce_population_count(m0)
                    pc1 = plsc.all_reduce_population_count(m1)
                    w0 = w0 + jnp.reshape(pc0[0:1], ())
                    w1 = w1 + jnp.reshape(pc1[0:1], ())
                return (w0, w1)

            return lax.fori_loop(0, 8, grp, ptrs)

        lax.fori_loop(0, PT_ROWS // 4, chunk,
                      (jnp.int32(0), jnp.int32(0)))
        pltpu.sync_copy(gb0.at[pl.ds(0, CAP)], gout.at[0, pl.ds(w * CAP, CAP)])
        pltpu.sync_copy(sb0.at[pl.ds(0, CAP)], sout.at[0, pl.ds(w * CAP, CAP)])
        pltpu.sync_copy(gb1.at[pl.ds(0, CAP)], gout.at[1, pl.ds(w * CAP, CAP)])
        pltpu.sync_copy(sb1.at[pl.ds(0, CAP)], sout.at[1, pl.ds(w * CAP, CAP)])

    run_dir(dstr, srcr, ga, sa)
    run_dir(srcr, dstr, gb, sb_)


_part = pl.kernel(
    _part_body,
    out_type=[jax.ShapeDtypeStruct((2, PT * CAP), jnp.int32)] * 4,
    mesh=_mesh,
    compiler_params=pltpu.CompilerParams(
        use_tc_tiling_on_sc=False, needs_layout_passes=False),
    scratch_types=[
        pltpu.VMEM((4, 128), jnp.int32),
        pltpu.VMEM((4, 128), jnp.int32),
        pltpu.VMEM((CAP + 16,), jnp.int32),
        pltpu.VMEM((CAP + 16,), jnp.int32),
        pltpu.VMEM((CAP + 16,), jnp.int32),
        pltpu.VMEM((CAP + 16,), jnp.int32),
    ],
)


def _seg5_body(tab, gidx, sidx, out, acc, ibufg, ibufs, rbuf, zbuf, gsem0, gsem1):
    c = lax.axis_index("c")
    s = lax.axis_index("s")

    def zrow(i, _):
        for q in range(4):
            zbuf[i, pl.ds(16 * q, 16)] = jnp.zeros((16,), jnp.float32)
        return 0

    lax.fori_loop(0, 128, zrow, 0)
    zbase = s * 1568

    def zcp(i, _):
        pltpu.sync_copy(zbuf, acc.at[pl.ds(zbase + i * 128, 128)])
        return 0

    lax.fori_loop(0, 12, zcp, 0)
    pltpu.sync_copy(zbuf.at[pl.ds(0, 32)], acc.at[pl.ds(zbase + 1536, 32)])
    plsc.subcore_barrier()

    row0 = s * TR5

    def sb(k, _):
        r0 = row0 + k * SB5
        pltpu.sync_copy(gidx.at[c, pl.ds(r0, SB5)], ibufg)
        pltpu.sync_copy(sidx.at[c, pl.ds(r0, SB5)], ibufs)
        pltpu.async_copy(tab.at[ibufg.at[0]], rbuf.at[0], gsem0)

        def ch2(t, _):
            j0 = 2 * t
            pltpu.async_copy(tab.at[ibufg.at[j0 + 1]], rbuf.at[1], gsem1)
            pltpu.make_async_copy(
                tab.at[ibufg.at[j0]], rbuf.at[0], gsem0).wait()
            pltpu.sync_copy(rbuf.at[0], acc.at[ibufs.at[j0]], add=True)

            @pl.when(t < SB5 // 2 - 1)
            def _():
                pltpu.async_copy(tab.at[ibufg.at[j0 + 2]], rbuf.at[0], gsem0)

            pltpu.make_async_copy(
                tab.at[ibufg.at[j0 + 1]], rbuf.at[1], gsem1).wait()
            pltpu.sync_copy(rbuf.at[1], acc.at[ibufs.at[j0 + 1]], add=True)
            return 0

        lax.fori_loop(0, SB5 // 2, ch2, 0)
        return 0

    lax.fori_loop(0, NSB5, sb, 0)
    plsc.subcore_barrier()

    @pl.when(s < 15)
    def _():
        pltpu.sync_copy(acc.at[pl.ds(s * 1568, 1568)],
                        out.at[pl.ds(c * HALF + s * 1568, 1568)])

    @pl.when(s == 15)
    def _():
        pltpu.sync_copy(acc.at[pl.ds(23520, 1504)],
                        out.at[pl.ds(c * HALF + 23520, 1504)])


_seg5 = pl.kernel(
    _seg5_body,
    out_type=jax.ShapeDtypeStruct((NP, D), jnp.float32),
    mesh=_mesh,
    compiler_params=pltpu.CompilerParams(use_tc_tiling_on_sc=False),
    scratch_types=[
        pltpu.VMEM_SHARED((ACC5, D), jnp.float32),
        pltpu.VMEM((SB5, 128), jnp.int32),
        pltpu.VMEM((SB5, 128), jnp.int32),
        pltpu.VMEM((2, 128, D), jnp.float32),
        pltpu.VMEM((128, D), jnp.float32),
        pltpu.SemaphoreType.DMA,
        pltpu.SemaphoreType.DMA,
    ],
)


def _deg_body(dsts, srcs, cnt, acc, ibuf, ones, zbuf):
    c = lax.axis_index("c")
    s = lax.axis_index("s")

    def fill(i, _):
        zbuf[i, pl.ds(0, 16)] = jnp.zeros((16,), jnp.float32)
        ones[i, pl.ds(0, 16)] = jnp.ones((16,), jnp.float32)
        return 0

    lax.fori_loop(0, 128, fill, 0)
    zbase = s * OPT

    def zcp(i, _):
        pltpu.sync_copy(zbuf, acc.at[pl.ds(zbase + i * 128, 128)])
        return 0

    lax.fori_loop(0, 24, zcp, 0)
    pltpu.sync_copy(zbuf.at[pl.ds(0, 56)], acc.at[pl.ds(zbase + 3072, 56)])
    plsc.subcore_barrier()

    row0 = s * TROWS

    def count(idx):
        def sb(k, _):
            pltpu.sync_copy(idx.at[pl.ds(row0 + k * SBD, SBD)], ibuf)

            def ch(j, _):
                pltpu.sync_copy(ones, acc.at[ibuf.at[j]], add=True)
                return 0

            lax.fori_loop(0, SBD, ch, 0)
            return 0

        lax.fori_loop(0, NSBD, sb, 0)

    @pl.when(c == 0)
    def _():
        count(dsts)

    @pl.when(c == 1)
    def _():
        count(srcs)

    plsc.subcore_barrier()
    ob = s * OPT
    pltpu.sync_copy(acc.at[pl.ds(ob, OPT)], cnt.at[c, pl.ds(ob, OPT)])


_deg = pl.kernel(
    _deg_body,
    out_type=jax.ShapeDtypeStruct((2, NP, 16), jnp.float32),
    mesh=_mesh,
    compiler_params=pltpu.CompilerParams(use_tc_tiling_on_sc=False),
    scratch_types=[
        pltpu.VMEM_SHARED((NP, 16), jnp.float32),
        pltpu.VMEM((SBD, 128), jnp.int32),
        pltpu.VMEM((128, 16), jnp.float32),
        pltpu.VMEM((128, 16), jnp.float32),
    ],
)


R = NP // 16
GB = NP // R


def _fc_body(x_ref, w_ref, b_ref, o_ref):
    o_ref[...] = (
        lax.dot_general(
            x_ref[...], w_ref[...], (((1,), (1,)), ((), ())),
            preferred_element_type=jnp.float32,
        )
        + b_ref[...]
    )


def _fc(item_pad, fc_w, fc_b):
    b2 = fc_b.reshape(1, D)
    return pl.pallas_call(
        _fc_body,
        grid=(GB,),
        in_specs=[
            pl.BlockSpec((R, D), lambda g: (g, 0)),
            pl.BlockSpec((D, D), lambda g: (0, 0)),
            pl.BlockSpec((1, D), lambda g: (0, 0)),
        ],
        out_specs=pl.BlockSpec((R, D), lambda g: (g, 0)),
        out_shape=jax.ShapeDtypeStruct((NP, D), jnp.float32),
    )(item_pad, fc_w, b2)


def _norm_body(acca, accb, cnti, cntu, sumi, sumu, hi, hu, soi, sou):
    rii = 1.0 / jnp.maximum(cnti[:, 0:1], 1.0)
    riu = 1.0 / jnp.maximum(cntu[:, 0:1], 1.0)
    new_i = acca[...] * rii
    new_u = accb[...] * riu
    hi[...] = new_i
    hu[...] = new_u
    soi[...] = sumi[...] + new_i
    sou[...] = sumu[...] + new_u


def _norm(acca, accb, cnti, cntu, sumi, sumu):
    blk = pl.BlockSpec((R, D), lambda g: (g, 0))
    cspec = pl.BlockSpec((R, 16), lambda g: (g, 0))
    return pl.pallas_call(
        _norm_body,
        grid=(GB,),
        in_specs=[blk, blk, cspec, cspec, blk, blk],
        out_specs=[blk, blk, blk, blk],
        out_shape=[jax.ShapeDtypeStruct((NP, D), jnp.float32)] * 4,
    )(acca, accb, cnti, cntu, sumi, sumu)


def _final_body(acca, accb, cnti, cntu, sumi, sumu, item_o, user_o):
    rii = 1.0 / jnp.maximum(cnti[:, 0:1], 1.0)
    riu = 1.0 / jnp.maximum(cntu[:, 0:1], 1.0)
    item_o[...] = (sumi[...] + acca[...] * rii) * 0.25
    user_o[...] = (sumu[...] + accb[...] * riu) * 0.25


def _final(acca, accb, cnti, cntu, sumi, sumu):
    blk = pl.BlockSpec((R, D), lambda g: (g, 0))
    cspec = pl.BlockSpec((R, 16), lambda g: (g, 0))
    return pl.pallas_call(
        _final_body,
        grid=(GB,),
        in_specs=[blk, blk, cspec, cspec, blk, blk],
        out_specs=[blk, blk],
        out_shape=[jax.ShapeDtypeStruct((NP, D), jnp.float32)] * 2,
    )(acca, accb, cnti, cntu, sumi, sumu)


def kernel(user_emb, item_emb, fc_w, fc_b, edge_index):
    src = edge_index[0].astype(jnp.int32)
    dst = edge_index[1].astype(jnp.int32)
    pad = EP - E
    padv = N + (jnp.arange(pad, dtype=jnp.int32) % (NP - N))
    src_p = jnp.concatenate([src, padv]).reshape(EP // 128, 128)
    dst_p = jnp.concatenate([dst, padv]).reshape(EP // 128, 128)

    cnt = _deg(dst_p, src_p)
    cnti = cnt[0]
    cntu = cnt[1]

    ga, sa, gb, sb_ = _part(src_p, dst_p)
    ga3 = ga.reshape(2, GROWS, 128)
    sa3 = sa.reshape(2, GROWS, 128)
    gb3 = gb.reshape(2, GROWS, 128)
    sb3 = sb_.reshape(2, GROWS, 128)

    user_pad = jnp.pad(user_emb, ((0, NP - N), (0, 0)))
    item_pad = jnp.pad(item_emb, ((0, NP - N), (0, 0)))

    hu = user_pad
    hi = _fc(item_pad, fc_w, fc_b)
    sumu = user_pad
    sumi = item_pad

    for layer in range(LAYERS):
        acca = _seg5(hu, ga3, sa3)
        accb = _seg5(hi, gb3, sb3)
        if layer < LAYERS - 1:
            hi, hu, sumi, sumu = _norm(acca, accb, cnti, cntu, sumi, sumu)
        else:
            item_out, user_out = _final(acca, accb, cnti, cntu, sumi, sumu)

    return (user_out[:N], item_out[:N])


# speedup vs baseline: 6.5488x; 6.5488x over previous
import jax
import jax.numpy as jnp
from jax import lax
from jax.experimental import pallas as pl
from jax.experimental.pallas import tpu as pltpu
from jax.experimental.pallas import tpu_sc as plsc

N = 50000
NP = 50048
HALF = NP // 2
ACC5 = 25088
D = 64
E = 1_000_000
EP = 1 << 20
PT = 32
PT_E = EP // PT
PT_ROWS = PT_E // 128
CAP = 17408
CROWS = CAP // 128
GROWS = PT * CROWS
TR5 = 2 * CROWS
SB5 = 16
NSB5 = TR5 // SB5
OPT = NP // 16
TROWS = (EP // 128) // 16
SBD = 16
NSBD = TROWS // SBD
LAYERS = 3

_mesh = plsc.VectorSubcoreMesh(core_axis_name="c", subcore_axis_name="s")


def _part_body(srcr, dstr, ga, sa, gb, sb_, bufk, bufv, gb0, sb0, gb1, sb1):
    c = lax.axis_index("c")
    s = lax.axis_index("s")
    w = c * 16 + s

    def run_dir(keyr, valr, gout, sout):
        def dfl(i, _):
            lanes = lax.iota(jnp.int32, 16)
            z = i * 16 + lanes
            t = HALF + ((i * 16 + lanes) % 64)
            gb0[pl.ds(i * 16, 16)] = z
            gb1[pl.ds(i * 16, 16)] = z
            sb0[pl.ds(i * 16, 16)] = t
            sb1[pl.ds(i * 16, 16)] = t
            return 0

        lax.fori_loop(0, CAP // 16, dfl, 0)

        def chunk(k, ptrs):
            pltpu.sync_copy(keyr.at[pl.ds(w * PT_ROWS + k * 4, 4)], bufk)
            pltpu.sync_copy(valr.at[pl.ds(w * PT_ROWS + k * 4, 4)], bufv)

            def grp(q, ptrs2):
                w0, w1 = ptrs2
                for r in range(4):
                    kv = bufk[r, pl.ds(16 * q, 16)]
                    vv = bufv[r, pl.ds(16 * q, 16)]
                    m0 = kv < HALF
                    m1 = jnp.logical_and(kv >= HALF, kv < N)
                    plsc.store_compressed(gb0.at[pl.ds(w0, 16)], vv, mask=m0)
                    plsc.store_compressed(sb0.at[pl.ds(w0, 16)], kv, mask=m0)
                    plsc.store_compressed(gb1.at[pl.ds(w1, 16)], vv, mask=m1)
                    plsc.store_compressed(
                        sb1.at[pl.ds(w1, 16)], kv - HALF, mask=m1)
                    pc0 = plsc.all_reduce_population_count(m0)
                    pc1 = plsc.all_reduce_population_count(m1)
                    w0 = w0 + jnp.reshape(pc0[0:1], ())
                    w1 = w1 + jnp.reshape(pc1[0:1], ())
                return (w0, w1)

            return lax.fori_loop(0, 8, grp, ptrs)

        lax.fori_loop(0, PT_ROWS // 4, chunk,
                      (jnp.int32(0), jnp.int32(0)))
        pltpu.sync_copy(gb0.at[pl.ds(0, CAP)], gout.at[0, pl.ds(w * CAP, CAP)])
        pltpu.sync_copy(sb0.at[pl.ds(0, CAP)], sout.at[0, pl.ds(w * CAP, CAP)])
        pltpu.sync_copy(gb1.at[pl.ds(0, CAP)], gout.at[1, pl.ds(w * CAP, CAP)])
        pltpu.sync_copy(sb1.at[pl.ds(0, CAP)], sout.at[1, pl.ds(w * CAP, CAP)])

    run_dir(dstr, srcr, ga, sa)
    run_dir(srcr, dstr, gb, sb_)


_part = pl.kernel(
    _part_body,
    out_type=[jax.ShapeDtypeStruct((2, PT * CAP), jnp.int32)] * 4,
    mesh=_mesh,
    compiler_params=pltpu.CompilerParams(
        use_tc_tiling_on_sc=False, needs_layout_passes=False),
    scratch_types=[
        pltpu.VMEM((4, 128), jnp.int32),
        pltpu.VMEM((4, 128), jnp.int32),
        pltpu.VMEM((CAP + 16,), jnp.int32),
        pltpu.VMEM((CAP + 16,), jnp.int32),
        pltpu.VMEM((CAP + 16,), jnp.int32),
        pltpu.VMEM((CAP + 16,), jnp.int32),
    ],
)


def _seg5_body(tab, gidx, sidx, out, acc, ibufg, ibufs, rbuf, zbuf, gsem0, gsem1):
    c = lax.axis_index("c")
    s = lax.axis_index("s")

    def zrow(i, _):
        for q in range(4):
            zbuf[i, pl.ds(16 * q, 16)] = jnp.zeros((16,), jnp.float32)
        return 0

    lax.fori_loop(0, 128, zrow, 0)
    zbase = s * 1568

    def zcp(i, _):
        pltpu.sync_copy(zbuf, acc.at[pl.ds(zbase + i * 128, 128)])
        return 0

    lax.fori_loop(0, 12, zcp, 0)
    pltpu.sync_copy(zbuf.at[pl.ds(0, 32)], acc.at[pl.ds(zbase + 1536, 32)])
    plsc.subcore_barrier()

    row0 = s * TR5

    def sb(k, _):
        r0 = row0 + k * SB5
        pltpu.sync_copy(gidx.at[c, pl.ds(r0, SB5)], ibufg)
        pltpu.sync_copy(sidx.at[c, pl.ds(r0, SB5)], ibufs)
        pltpu.async_copy(tab.at[ibufg.at[0]], rbuf.at[0], gsem0)

        def ch2(t, _):
            j0 = 2 * t
            pltpu.async_copy(tab.at[ibufg.at[j0 + 1]], rbuf.at[1], gsem1)
            pltpu.make_async_copy(
                tab.at[ibufg.at[j0]], rbuf.at[0], gsem0).wait()
            pltpu.sync_copy(rbuf.at[0], acc.at[ibufs.at[j0]], add=True)

            @pl.when(t < SB5 // 2 - 1)
            def _():
                pltpu.async_copy(tab.at[ibufg.at[j0 + 2]], rbuf.at[0], gsem0)

            pltpu.make_async_copy(
                tab.at[ibufg.at[j0 + 1]], rbuf.at[1], gsem1).wait()
            pltpu.sync_copy(rbuf.at[1], acc.at[ibufs.at[j0 + 1]], add=True)
            return 0

        lax.fori_loop(0, SB5 // 2, ch2, 0)
        return 0

    lax.fori_loop(0, NSB5, sb, 0)
    plsc.subcore_barrier()

    @pl.when(s < 15)
    def _():
        pltpu.sync_copy(acc.at[pl.ds(s * 1568, 1568)],
                        out.at[pl.ds(c * HALF + s * 1568, 1568)])

    @pl.when(s == 15)
    def _():
        pltpu.sync_copy(acc.at[pl.ds(23520, 1504)],
                        out.at[pl.ds(c * HALF + 23520, 1504)])


_seg5 = pl.kernel(
    _seg5_body,
    out_type=jax.ShapeDtypeStruct((NP, D), jnp.float32),
    mesh=_mesh,
    compiler_params=pltpu.CompilerParams(use_tc_tiling_on_sc=False),
    scratch_types=[
        pltpu.VMEM_SHARED((ACC5, D), jnp.float32),
        pltpu.VMEM((SB5, 128), jnp.int32),
        pltpu.VMEM((SB5, 128), jnp.int32),
        pltpu.VMEM((2, 128, D), jnp.float32),
        pltpu.VMEM((128, D), jnp.float32),
        pltpu.SemaphoreType.DMA,
        pltpu.SemaphoreType.DMA,
    ],
)


def _deg_body(dsts, srcs, cnt, acc, ibuf, ones, zbuf):
    c = lax.axis_index("c")
    s = lax.axis_index("s")

    def fill(i, _):
        zbuf[i, pl.ds(0, 16)] = jnp.zeros((16,), jnp.float32)
        ones[i, pl.ds(0, 16)] = jnp.ones((16,), jnp.float32)
        return 0

    lax.fori_loop(0, 128, fill, 0)
    zbase = s * OPT

    def zcp(i, _):
        pltpu.sync_copy(zbuf, acc.at[pl.ds(zbase + i * 128, 128)])
        return 0

    lax.fori_loop(0, 24, zcp, 0)
    pltpu.sync_copy(zbuf.at[pl.ds(0, 56)], acc.at[pl.ds(zbase + 3072, 56)])
    plsc.subcore_barrier()

    row0 = s * TROWS

    def count(idx):
        def sb(k, _):
            pltpu.sync_copy(idx.at[pl.ds(row0 + k * SBD, SBD)], ibuf)

            def ch(j, _):
                pltpu.sync_copy(ones, acc.at[ibuf.at[j]], add=True)
                return 0

            lax.fori_loop(0, SBD, ch, 0)
            return 0

        lax.fori_loop(0, NSBD, sb, 0)

    @pl.when(c == 0)
    def _():
        count(dsts)

    @pl.when(c == 1)
    def _():
        count(srcs)

    plsc.subcore_barrier()
    ob = s * OPT
    pltpu.sync_copy(acc.at[pl.ds(ob, OPT)], cnt.at[c, pl.ds(ob, OPT)])


_deg = pl.kernel(
    _deg_body,
    out_type=jax.ShapeDtypeStruct((2, NP, 16), jnp.float32),
    mesh=_mesh,
    compiler_params=pltpu.CompilerParams(use_tc_tiling_on_sc=False),
    scratch_types=[
        pltpu.VMEM_SHARED((NP, 16), jnp.float32),
        pltpu.VMEM((SBD, 128), jnp.int32),
        pltpu.VMEM((128, 16), jnp.float32),
        pltpu.VMEM((128, 16), jnp.float32),
    ],
)


R = NP // 16
GB = NP // R


def _fc_body(x_ref, w_ref, b_ref, o_ref):
    o_ref[...] = (
        lax.dot_general(
            x_ref[...], w_ref[...], (((1,), (1,)), ((), ())),
            preferred_element_type=jnp.float32,
        )
        + b_ref[...]
    )


def _fc(item_pad, fc_w, fc_b):
    b2 = fc_b.reshape(1, D)
    return pl.pallas_call(
        _fc_body,
        grid=(GB,),
        in_specs=[
            pl.BlockSpec((R, D), lambda g: (g, 0)),
            pl.BlockSpec((D, D), lambda g: (0, 0)),
            pl.BlockSpec((1, D), lambda g: (0, 0)),
        ],
        out_specs=pl.BlockSpec((R, D), lambda g: (g, 0)),
        out_shape=jax.ShapeDtypeStruct((NP, D), jnp.float32),
    )(item_pad, fc_w, b2)


def _norm_body(acca, accb, cnti, cntu, sumi, sumu, hi, hu, soi, sou):
    rii = 1.0 / jnp.maximum(cnti[:, 0:1], 1.0)
    riu = 1.0 / jnp.maximum(cntu[:, 0:1], 1.0)
    new_i = acca[...] * rii
    new_u = accb[...] * riu
    hi[...] = new_i
    hu[...] = new_u
    soi[...] = sumi[...] + new_i
    sou[...] = sumu[...] + new_u


def _norm(acca, accb, cnti, cntu, sumi, sumu):
    blk = pl.BlockSpec((R, D), lambda g: (g, 0))
    cspec = pl.BlockSpec((R, 16), lambda g: (g, 0))
    return pl.pallas_call(
        _norm_body,
        grid=(GB,),
        in_specs=[blk, blk, cspec, cspec, blk, blk],
        out_specs=[blk, blk, blk, blk],
        out_shape=[jax.ShapeDtypeStruct((NP, D), jnp.float32)] * 4,
    )(acca, accb, cnti, cntu, sumi, sumu)


def _final_body(acca, accb, cnti, cntu, sumi, sumu, item_o, user_o):
    rii = 1.0 / jnp.maximum(cnti[:, 0:1], 1.0)
    riu = 1.0 / jnp.maximum(cntu[:, 0:1], 1.0)
    item_o[...] = (sumi[...] + acca[...] * rii) * 0.25
    user_o[...] = (sumu[...] + accb[...] * riu) * 0.25


def _final(acca, accb, cnti, cntu, sumi, sumu):
    blk = pl.BlockSpec((R, D), lambda g: (g, 0))
    cspec = pl.BlockSpec((R, 16), lambda g: (g, 0))
    return pl.pallas_call(
        _final_body,
        grid=(GB,),
        in_specs=[blk, blk, cspec, cspec, blk, blk],
        out_specs=[blk, blk],
        out_shape=[jax.ShapeDtypeStruct((NP, D), jnp.float32)] * 2,
    )(acca, accb, cnti, cntu, sumi, sumu)


def kernel(user_emb, item_emb, fc_w, fc_b, edge_index):
    src = edge_index[0].astype(jnp.int32)
    dst = edge_index[1].astype(jnp.int32)
    pad = EP - E
    padv = N + (jnp.arange(pad, dtype=jnp.int32) % (NP - N))
    src_p = jnp.concatenate([src, padv]).reshape(EP // 128, 128)
    dst_p = jnp.concatenate([dst, padv]).reshape(EP // 128, 128)

    cnt = _deg(dst_p, src_p)
    cnti = cnt[0]
    cntu = cnt[1]

    ga, sa, gb, sb_ = _part(src_p, dst_p)
    ga3 = ga.reshape(2, GROWS, 128)
    sa3 = sa.reshape(2, GROWS, 128)
    gb3 = gb.reshape(2, GROWS, 128)
    sb3 = sb_.reshape(2, GROWS, 128)

    user_pad = jnp.pad(user_emb, ((0, NP - N), (0, 0)))
    item_pad = jnp.pad(item_emb, ((0, NP - N), (0, 0)))

    hu = user_pad
    hi = _fc(item_pad, fc_w, fc_b)
    sumu = user_pad
    sumi = item_pad

    for layer in range(LAYERS):
        acca = _seg5(hu, ga3, sa3)
        accb = _seg5(hi, gb3, sb3)
        if layer < LAYERS - 1:
            hi, hu, sumi, sumu = _norm(acca, accb, cnti, cntu, sumi, sumu)
        else:
            item_out, user_out = _final(acca, accb, cnti, cntu, sumi, sumu)

    return (user_out[:N], item_out[:N])
